# baseline (speedup 1.0000x reference)
import functools

import jax
import jax.numpy as jnp
from jax import lax
from jax.experimental import pallas as pl
from jax.experimental.pallas import tpu as pltpu
from jax.experimental.pallas import tpu_sc as plsc

NC = 2
NS = 16
L = 16
NW = NC * NS

F32 = jnp.float32
I32 = jnp.int32


def _splat(vec, j):
    dnums = lax.GatherDimensionNumbers(
        offset_dims=(), collapsed_slice_dims=(0,), start_index_map=(0,))
    return lax.gather(vec, jnp.full((L, 1), j, I32), dnums, slice_sizes=(1,),
                      mode=lax.GatherScatterMode.PROMISE_IN_BOUNDS)

def _bucket_body(R, EP, EPW, CAPW, SLICE, MAG, SH,
                 src_hbm, dst_hbm, ew_hbm, relpad_hbm,
                 bfl_hbm, bwm_hbm, bds_hbm, cnts_hbm,
                 src_v, ds_v, ew_v, fl_all, wm_all, pb_all,
                 st_fl, st_wm, st_ds, rel_v, cv_v):
    cid = lax.axis_index("c")
    sid = lax.axis_index("s")
    wid = cid * NS + sid
    ebase = wid * EPW
    NV = EPW // L

    pltpu.sync_copy(relpad_hbm, rel_v)
    pltpu.sync_copy(src_hbm.at[pl.ds(ebase, EPW)], src_v)
    pltpu.sync_copy(dst_hbm.at[pl.ds(ebase, EPW)], ds_v)
    pltpu.sync_copy(ew_hbm.at[pl.ds(ebase, EPW)], ew_v)
    relvec = rel_v[...]
    srel = [_splat(relvec, r) for r in range(R + 1)]
    iota = lax.iota(I32, L)

    def vec(v, carry):
        off = v * L
        sv = src_v[pl.ds(off, L)]
        dv = ds_v[pl.ds(off, L)]
        wv = ew_v[pl.ds(off, L)]
        pos = (ebase + off) + iota
        rid = jnp.full((L,), -1, I32)
        for r in range(R):
            rid = rid + jnp.where(pos >= srel[r], 1, 0).astype(I32)
        valid = (rid >= 0) & (pos < srel[R])
        fl_all[pl.ds(off, L)] = jnp.where(valid, sv * R + rid,
                                          jnp.zeros((L,), I32))
        wm_all[pl.ds(off, L)] = jnp.where(valid, wv, jnp.zeros((L,), F32))
        pb_all[pl.ds(off, L)] = lax.shift_right_logical(dv * MAG, SH)
        return carry

    lax.fori_loop(0, NV, vec, 0)

    cnt_vec = jnp.zeros((L,), I32)
    zero16 = jnp.zeros((L,), I32)
    for p in range(NS):
        def coll(v, off_vec):
            off16 = v * L
            m = pb_all[pl.ds(off16, L)] == p
            mi = m.astype(I32)
            csum = plsc.cumsum(mi)
            idx = jnp.maximum(off_vec + csum - 1, zero16)
            plsc.store_scatter(st_fl, (idx,), fl_all[pl.ds(off16, L)],
                               mask=m)
            plsc.store_scatter(st_wm, (idx,), wm_all[pl.ds(off16, L)],
                               mask=m)
            plsc.store_scatter(st_ds, (idx,), ds_v[pl.ds(off16, L)],
                               mask=m)
            return off_vec + _splat(csum, L - 1)

        off_vec = lax.fori_loop(0, NV, coll, zero16)
        dfl = jnp.zeros((L,), I32)
        dwm = jnp.zeros((L,), F32)
        dds = jnp.full((L,), p * SLICE, I32)
        for q in range(4):
            di = off_vec + (iota + q * L)
            plsc.store_scatter(st_fl, (di,), dfl)
            plsc.store_scatter(st_wm, (di,), dwm)
            plsc.store_scatter(st_ds, (di,), dds)
        off2 = lax.shift_left(lax.shift_right_logical(off_vec + 63, 6), 6)
        cnt_vec = cnt_vec + jnp.where(iota == p, off2, zero16)
        rbase = (wid * NS + p) * CAPW
        pltpu.sync_copy(st_fl.at[pl.ds(0, CAPW)], bfl_hbm.at[pl.ds(rbase, CAPW)])
        pltpu.sync_copy(st_wm.at[pl.ds(0, CAPW)], bwm_hbm.at[pl.ds(rbase, CAPW)])
        pltpu.sync_copy(st_ds.at[pl.ds(0, CAPW)], bds_hbm.at[pl.ds(rbase, CAPW)])

    cv_v[...] = cnt_vec
    pltpu.sync_copy(cv_v, cnts_hbm.at[wid])


def _make_bucket(R, EP, SLICE):
    EPW = EP // NW
    CAPW = EPW + 128
    STC = CAPW + L
    assert EPW % L == 0 and CAPW % 8 == 0
    mesh = plsc.VectorSubcoreMesh(core_axis_name="c", subcore_axis_name="s",
                                  num_cores=NC, num_subcores=NS)
    return pl.kernel(
        functools.partial(_bucket_body, R, EP, EPW, CAPW, SLICE, 3277, 21),
        out_type=(jax.ShapeDtypeStruct((NW * NS * CAPW,), I32),
                  jax.ShapeDtypeStruct((NW * NS * CAPW,), F32),
                  jax.ShapeDtypeStruct((NW * NS * CAPW,), I32),
                  jax.ShapeDtypeStruct((NW, L), I32)),
        mesh=mesh,
        compiler_params=pltpu.CompilerParams(needs_layout_passes=False),
        scratch_types=[
            pltpu.VMEM((EPW,), I32),
            pltpu.VMEM((EPW,), I32),
            pltpu.VMEM((EPW,), F32),
            pltpu.VMEM((EPW,), I32),
            pltpu.VMEM((EPW,), F32),
            pltpu.VMEM((EPW,), I32),
            pltpu.VMEM((STC,), I32),
            pltpu.VMEM((STC,), F32),
            pltpu.VMEM((STC,), I32),
            pltpu.VMEM((L,), I32),
            pltpu.VMEM((L,), I32),
        ],
    )


def _edge_body(NP, H, CAPW, C, table_hbm, bfl_hbm, bwm_hbm, bds_hbm,
               cnts_hbm, out_hbm, *refs):
    ifb = refs[0:4]
    wvb = refs[4:8]
    dvb = refs[8:12]
    rows = refs[12:16]
    acc = refs[16]
    cnv = refs[17]
    semm = refs[18:22]
    semg = refs[22:26]
    cid = lax.axis_index("c")
    sid = lax.axis_index("s")
    SLICE = NP // NS
    iota = lax.iota(I32, L)
    zv = jnp.zeros((L,), F32)

    def zrow(j, c):
        for k in range(H // L):
            acc[j, pl.ds(k * L, L)] = zv
        return c
    lax.fori_loop(0, SLICE, zrow, 0)

    pltpu.sync_copy(cnts_hbm, cnv)
    rv = cnv[sid, pl.ds(cid * NS, L)]
    cums = []
    tot = jnp.int32(0)
    for j in range(NS):
        cums.append(tot)
        tot = tot + lax.shift_right_logical(rv[j], 6)

    def locate(t):
        rj = jnp.int32(0)
        for j in range(1, NS):
            rj = rj + (t >= cums[j]).astype(I32)
        cum_rj = jnp.int32(0)
        for j in range(NS):
            cum_rj = cum_rj + jnp.where(rj == j, cums[j], jnp.int32(0))
        region = (cid * NS + rj) * NS + sid
        return region * CAPW + (t - cum_rj) * C

    def issue_smalls(t, slot):
        base = locate(t)
        pltpu.async_copy(bfl_hbm.at[pl.ds(base, C)], ifb[slot],
                         semm[slot])
        pltpu.async_copy(bwm_hbm.at[pl.ds(base, C)], wvb[slot],
                         semm[slot])
        pltpu.async_copy(bds_hbm.at[pl.ds(base, C)], dvb[slot],
                         semm[slot])

    def wait_smalls(slot):
        pltpu.make_async_copy(bfl_hbm.at[pl.ds(0, C)], ifb[slot],
                              semm[slot]).wait()
        pltpu.make_async_copy(bwm_hbm.at[pl.ds(0, C)], wvb[slot],
                              semm[slot]).wait()
        pltpu.make_async_copy(bds_hbm.at[pl.ds(0, C)], dvb[slot],
                              semm[slot]).wait()

    for k in range(3):
        @pl.when(k < tot)
        def _(k=k):
            issue_smalls(jnp.int32(k), k)
    for k in range(2):
        @pl.when(k < tot)
        def _(k=k):
            wait_smalls(k)
            pltpu.async_copy(table_hbm.at[ifb[k]], rows[k], semg[k])

    s640 = sid * SLICE

    def grp4(gq, carry):
        for b in range(4):
            t = gq * 4 + b

            @pl.when(t + 2 < tot)
            def _():
                wait_smalls((b + 2) % 4)
                pltpu.async_copy(table_hbm.at[ifb[(b + 2) % 4]],
                                 rows[(b + 2) % 4], semg[(b + 2) % 4])

            @pl.when(t + 3 < tot)
            def _():
                issue_smalls(t + 3, (b + 3) % 4)

            @pl.when(t < tot)
            def _():
                pltpu.make_async_copy(table_hbm.at[ifb[b]], rows[b],
                                      semg[b]).wait()

                @plsc.parallel_loop(0, C // L)
                def grp(g):
                    wvec = wvb[b][pl.ds(g * L, L)]
                    dvec = dvb[b][pl.ds(g * L, L)] - s640
                    for j in range(L):
                        wj = _splat(wvec, j)
                        dl = dvec[j]
                        e = g * L + j
                        prods = [rows[b][e, pl.ds(k * L, L)] * wj
                                 for k in range(H // L)]
                        for k in range(H // L):
                            plsc.addupdate(acc.at[dl, pl.ds(k * L, L)],
                                           prods[k])
        return carry

    lax.fori_loop(0, lax.shift_right_logical(tot + 3, 2), grp4, 0)
    pltpu.sync_copy(acc,
                    out_hbm.at[pl.ds(cid * NP + sid * SLICE, SLICE)])


def _make_edge(NP, H, CAPW):
    C = 64
    SLICE = NP // NS
    assert NP % NS == 0 and SLICE % 8 == 0 and CAPW % C == 0
    mesh = plsc.VectorSubcoreMesh(core_axis_name="c", subcore_axis_name="s",
                                  num_cores=NC, num_subcores=NS)
    return pl.kernel(
        functools.partial(_edge_body, NP, H, CAPW, C),
        out_type=jax.ShapeDtypeStruct((NC * NP, H), F32),
        mesh=mesh,
        scratch_types=(
            [pltpu.VMEM((C,), I32) for _ in range(4)]
            + [pltpu.VMEM((C,), F32) for _ in range(4)]
            + [pltpu.VMEM((C,), I32) for _ in range(4)]
            + [pltpu.VMEM((C, H), F32) for _ in range(4)]
            + [pltpu.VMEM((SLICE, H), F32)]
            + [pltpu.VMEM((L, NW), I32)]
            + [pltpu.SemaphoreType.DMA for _ in range(8)]
        ),
    )


def _tgather_body(H, NTP, TPW, CT, tidx_hbm, self_hbm, p0_hbm, p1_hbm,
                  out_hbm, idx_v, a_v, b_v, c_v, sem):
    cid = lax.axis_index("c")
    sid = lax.axis_index("s")
    wid = cid * NS + sid
    tbase = wid * TPW

    def chunk(ci, carry):
        base = tbase + ci * CT
        pltpu.sync_copy(tidx_hbm.at[pl.ds(base, CT)], idx_v)
        ca = pltpu.async_copy(self_hbm.at[idx_v], a_v, sem)
        cb = pltpu.async_copy(p0_hbm.at[idx_v], b_v, sem)
        cc = pltpu.async_copy(p1_hbm.at[idx_v], c_v, sem)
        ca.wait()
        cb.wait()
        cc.wait()

        def row(j, c2):
            for k in range(H // L):
                s = pl.ds(k * L, L)
                a_v[j, s] = jnp.maximum(a_v[j, s] + b_v[j, s] + c_v[j, s],
                                        jnp.zeros((L,), F32))
            return c2

        lax.fori_loop(0, CT, row, 0)
        pltpu.sync_copy(a_v, out_hbm.at[pl.ds(base, CT)])
        return carry

    lax.fori_loop(0, TPW // CT, chunk, 0)


def _make_tgather(N, H, NTP):
    TPW = NTP // NW
    CT = 80
    assert TPW % CT == 0
    mesh = plsc.VectorSubcoreMesh(core_axis_name="c", subcore_axis_name="s",
                                  num_cores=NC, num_subcores=NS)
    return pl.kernel(
        functools.partial(_tgather_body, H, NTP, TPW, CT),
        out_type=jax.ShapeDtypeStruct((NTP, H), F32),
        mesh=mesh,
        scratch_types=[
            pltpu.VMEM((CT,), I32),
            pltpu.VMEM((CT, H), F32),
            pltpu.VMEM((CT, H), F32),
            pltpu.VMEM((CT, H), F32),
            pltpu.SemaphoreType.DMA,
        ],
    )


def _expand_front_input(NTYPES, refs):
    (nt_ref, x_ref, wxT_ref, wtT_ref, temb_ref, bin_ref) = refs
    tb = jnp.dot(temb_ref[...], wtT_ref[...], preferred_element_type=F32)
    nt = nt_ref[...]
    tsel = jnp.zeros((nt.shape[0], tb.shape[1]), F32)
    for t in range(NTYPES):
        tsel = tsel + jnp.where(nt == t, 1.0, 0.0) * tb[t][None, :]
    x = x_ref[...]
    return jnp.maximum(
        jnp.dot(x, wxT_ref[...], preferred_element_type=F32)
        + tsel + bin_ref[...], 0.0)


def _expand_body_common(R, rel_ref, h, selfW_ref, selfb_ref, W2_ref,
                        table_ref, selfp_ref):
    table_ref[...] = jnp.dot(h, W2_ref[...], preferred_element_type=F32)
    Wc = jnp.zeros(selfW_ref.shape[1:], F32)
    bc = jnp.zeros((1, selfb_ref.shape[1]), F32)
    for r in range(R):
        act = rel_ref[r + 1] > rel_ref[r]
        Wc = Wc + jnp.where(act, selfW_ref[r], 0.0)
        bc = bc + jnp.where(act, selfb_ref[r][None, :], 0.0)
    selfp_ref[...] = lax.dot_general(
        h, Wc, (((1,), (1,)), ((), ())), preferred_element_type=F32) + bc


def _tca_body(R, NTYPES, rel_ref, nt_ref, x_ref, wxT_ref, wtT_ref, temb_ref,
              bin_ref, selfW_ref, selfb_ref, W2_ref, table_ref, selfp_ref):
    h = _expand_front_input(
        NTYPES, (nt_ref, x_ref, wxT_ref, wtT_ref, temb_ref, bin_ref))
    _expand_body_common(R, rel_ref, h, selfW_ref, selfb_ref, W2_ref,
                        table_ref, selfp_ref)


def _tcb_body(R, rel_ref, sp_ref, p0_ref, p1_ref, selfW_ref, selfb_ref,
              W2_ref, table_ref, selfp_ref):
    h = jnp.maximum(sp_ref[...] + p0_ref[...] + p1_ref[...], 0.0)
    _expand_body_common(R, rel_ref, h, selfW_ref, selfb_ref, W2_ref,
                        table_ref, selfp_ref)


def _tcd_body(ht_ref, w1_ref, b1_ref, w2_ref, b2_ref, out_ref):
    m = jnp.maximum(
        jnp.dot(ht_ref[...], w1_ref[...], preferred_element_type=F32)
        + b1_ref[...], 0.0)
    out_ref[...] = jnp.dot(m, w2_ref[...], preferred_element_type=F32) \
        + b2_ref[...]


def kernel(x, edge_index, edge_weight, rel_ptr, node_type, target_index,
           type_emb, lin_in_W, lin_in_b, l1_self_W, l1_self_b, l1_nei_W,
           l2_self_W, l2_self_b, l2_nei_W, mlp1_W, mlp1_b, mlp2_W, mlp2_b):
    N, D = x.shape
    E = edge_index.shape[1]
    R, H, _ = l1_self_W.shape
    NTYPES, TE = type_emb.shape
    OUT = mlp2_W.shape[0]
    NT = target_index.shape[0]

    EP = ((E + NW * 512 - 1) // (NW * 512)) * (NW * 512)
    NTP = ((NT + NW * 80 - 1) // (NW * 80)) * (NW * 80)

    src = jnp.concatenate([edge_index[0], jnp.zeros((EP - E,), I32)])
    dstp = jnp.concatenate([edge_index[1], jnp.zeros((EP - E,), I32)])
    ewp = jnp.concatenate([edge_weight, jnp.zeros((EP - E,), F32)])
    relpad = jnp.concatenate([rel_ptr, jnp.zeros((16 - (R + 1),), I32)])
    tidxp = jnp.concatenate([target_index, jnp.zeros((NTP - NT,), I32)])
    nt2 = node_type.reshape(N, 1)
    wxT = lin_in_W[:, :D].T
    wtT = lin_in_W[:, D:].T
    bin2 = lin_in_b.reshape(1, H)
    W2_1 = l1_nei_W.transpose(2, 0, 1).reshape(H, R * H)
    W2_2 = l2_nei_W.transpose(2, 0, 1).reshape(H, R * H)
    w1T = mlp1_W.T
    b1r = mlp1_b.reshape(1, H)
    w2T = mlp2_W.T
    b2r = mlp2_b.reshape(1, OUT)

    NPAD = ((N + NS * 128 - 1) // (NS * 128)) * (NS * 128)
    SLICE = NPAD // NS
    bfl, bwm, bds, cnts = _make_bucket(R, EP, SLICE)(src, dstp, ewp, relpad)

    BN = 1000
    grid = (N // BN,)
    tca = pl.pallas_call(
        functools.partial(_tca_body, R, NTYPES),
        grid=grid,
        in_specs=[
            pl.BlockSpec(memory_space=pltpu.SMEM),
            pl.BlockSpec((BN, 1), lambda i: (i, 0)),
            pl.BlockSpec((BN, D), lambda i: (i, 0)),
            pl.BlockSpec((D, H), lambda i: (0, 0)),
            pl.BlockSpec((TE, H), lambda i: (0, 0)),
            pl.BlockSpec((NTYPES, TE), lambda i: (0, 0)),
            pl.BlockSpec((1, H), lambda i: (0, 0)),
            pl.BlockSpec((R, H, H), lambda i: (0, 0, 0)),
            pl.BlockSpec((R, H), lambda i: (0, 0)),
            pl.BlockSpec((H, R * H), lambda i: (0, 0)),
        ],
        out_specs=[
            pl.BlockSpec((BN, R * H), lambda i: (i, 0)),
            pl.BlockSpec((BN, H), lambda i: (i, 0)),
        ],
        out_shape=[
            jax.ShapeDtypeStruct((N, R * H), F32),
            jax.ShapeDtypeStruct((N, H), F32),
        ],
    )
    table1, self1 = tca(rel_ptr, nt2, x, wxT, wtT, type_emb, bin2,
                        l1_self_W, l1_self_b, W2_1)

    edge_k = _make_edge(NPAD, H, EP // NW + 128)
    cntsT = cnts.T
    parts1 = edge_k(table1.reshape(N * R, H), bfl, bwm, bds, cntsT)
    p1a, p1b = parts1[:N], parts1[NPAD:NPAD + N]

    tcb = pl.pallas_call(
        functools.partial(_tcb_body, R),
        grid=grid,
        in_specs=[
            pl.BlockSpec(memory_space=pltpu.SMEM),
            pl.BlockSpec((BN, H), lambda i: (i, 0)),
            pl.BlockSpec((BN, H), lambda i: (i, 0)),
            pl.BlockSpec((BN, H), lambda i: (i, 0)),
            pl.BlockSpec((R, H, H), lambda i: (0, 0, 0)),
            pl.BlockSpec((R, H), lambda i: (0, 0)),
            pl.BlockSpec((H, R * H), lambda i: (0, 0)),
        ],
        out_specs=[
            pl.BlockSpec((BN, R * H), lambda i: (i, 0)),
            pl.BlockSpec((BN, H), lambda i: (i, 0)),
        ],
        out_shape=[
            jax.ShapeDtypeStruct((N, R * H), F32),
            jax.ShapeDtypeStruct((N, H), F32),
        ],
    )
    table2, self2 = tcb(rel_ptr, self1, p1a, p1b,
                        l2_self_W, l2_self_b, W2_2)

    parts2 = edge_k(table2.reshape(N * R, H), bfl, bwm, bds, cntsT)
    p2a, p2b = parts2[:N], parts2[NPAD:NPAD + N]

    htp = _make_tgather(N, H, NTP)(tidxp, self2, p2a, p2b)
    ht = htp[:NT]

    BM = 200
    tcd = pl.pallas_call(
        _tcd_body,
        grid=(NT // BM,),
        in_specs=[
            pl.BlockSpec((BM, H), lambda i: (i, 0)),
            pl.BlockSpec((H, H), lambda i: (0, 0)),
            pl.BlockSpec((1, H), lambda i: (0, 0)),
            pl.BlockSpec((H, OUT), lambda i: (0, 0)),
            pl.BlockSpec((1, OUT), lambda i: (0, 0)),
        ],
        out_specs=pl.BlockSpec((BM, OUT), lambda i: (i, 0)),
        out_shape=jax.ShapeDtypeStruct((NT, OUT), F32),
    )
    return tcd(ht, w1T, b1r, w2T, b2r)


# speedup vs baseline: 1.9526x; 1.9526x over previous
import functools

import jax
import jax.numpy as jnp
from jax import lax
from jax.experimental import pallas as pl
from jax.experimental.pallas import tpu as pltpu
from jax.experimental.pallas import tpu_sc as plsc

NC = 2
NS = 16
L = 16
NW = NC * NS

F32 = jnp.float32
I32 = jnp.int32


def _splat(vec, j):
    dnums = lax.GatherDimensionNumbers(
        offset_dims=(), collapsed_slice_dims=(0,), start_index_map=(0,))
    return lax.gather(vec, jnp.full((L, 1), j, I32), dnums, slice_sizes=(1,),
                      mode=lax.GatherScatterMode.PROMISE_IN_BOUNDS)

def _prep_body(R, EP, EPW, CP, src_hbm, dst_hbm, ew_hbm, relpad_hbm,
               meta_hbm, src_v, ds_v, w_v, rel_v, st_m):
    cid = lax.axis_index("c")
    sid = lax.axis_index("s")
    wid = cid * NS + sid
    ebase = wid * EPW
    NBC = CP // 64
    pltpu.sync_copy(relpad_hbm, rel_v)
    relvec = rel_v[...]
    srel = [_splat(relvec, r) for r in range(R + 1)]
    iota = lax.iota(I32, L)

    def chunk(ci, carry):
        base = ebase + ci * CP
        pltpu.sync_copy(src_hbm.at[pl.ds(base, CP)], src_v)
        pltpu.sync_copy(dst_hbm.at[pl.ds(base, CP)], ds_v)
        pltpu.sync_copy(ew_hbm.at[pl.ds(base, CP)], w_v)

        def grp(g, c2):
            off = g * L
            sv = src_v[pl.ds(off, L)]
            dv = ds_v[pl.ds(off, L)]
            wv = w_v[pl.ds(off, L)]
            pos = (base + off) + iota
            rid = jnp.full((L,), -1, I32)
            for r in range(R):
                rid = rid + jnp.where(pos >= srel[r], 1, 0).astype(I32)
            valid = (rid >= 0) & (pos < srel[R])
            fl = jnp.where(valid, sv * R + rid, jnp.zeros((L,), I32))
            wmv = jnp.where(valid, wv, jnp.zeros((L,), F32))
            gb = lax.shift_right_logical(g, 2)
            lo = (g & 3) * L
            st_m[pl.ds(gb * 192 + lo, L)] = fl
            st_m[pl.ds(gb * 192 + 64 + lo, L)] = plsc.bitcast(wmv, I32)
            st_m[pl.ds(gb * 192 + 128 + lo, L)] = dv
            return c2

        lax.fori_loop(0, CP // L, grp, 0)
        pltpu.sync_copy(st_m,
                        meta_hbm.at[pl.ds((wid * (EPW // 64) + ci * NBC) * 192,
                                          NBC * 192)])
        return carry

    lax.fori_loop(0, EPW // CP, chunk, 0)


def _make_prep(R, EP):
    EPW = EP // NW
    CP = 512
    assert EPW % CP == 0 and CP % 64 == 0
    mesh = plsc.VectorSubcoreMesh(core_axis_name="c", subcore_axis_name="s",
                                  num_cores=NC, num_subcores=NS)
    return pl.kernel(
        functools.partial(_prep_body, R, EP, EPW, CP),
        out_type=jax.ShapeDtypeStruct(((EP // 64) * 192,), I32),
        mesh=mesh,
        compiler_params=pltpu.CompilerParams(needs_layout_passes=False),
        scratch_types=[
            pltpu.VMEM((CP,), I32),
            pltpu.VMEM((CP,), I32),
            pltpu.VMEM((CP,), F32),
            pltpu.VMEM((L,), I32),
            pltpu.VMEM(((CP // 64) * 192,), I32),
        ],
    )


def _edge_body(NP, H, EP, EPW, C, table_hbm, meta_hbm, out_hbm, *refs):
    mb = refs[0:8]
    dvx = refs[8:16]
    rows = refs[16:20]
    acc_sh = refs[20]
    semm = refs[21:29]
    semg = refs[29:33]
    sems = refs[33:37]
    cid = lax.axis_index("c")
    sid = lax.axis_index("s")
    wid = cid * NS + sid
    NPR = NP // NS
    NCH = EPW // C
    bbase = wid * NCH

    def issue_meta(ci, slot):
        pltpu.async_copy(meta_hbm.at[pl.ds((bbase + ci) * 192, 192)],
                         mb[slot], semm[slot])

    def wait_meta(slot):
        pltpu.make_async_copy(meta_hbm.at[pl.ds(0, 192)], mb[slot],
                              semm[slot]).wait()

    for c0 in range(3):
        issue_meta(c0, c0)

    def zrow(j, c):
        for k in range(H // L):
            rows[0][j, pl.ds(k * L, L)] = jnp.zeros((L,), F32)
        return c
    lax.fori_loop(0, C, zrow, 0)
    for q in range(NPR // C):
        pltpu.sync_copy(rows[0], acc_sh.at[pl.ds(sid * NPR + q * C, C)])
    plsc.subcore_barrier()

    wait_meta(0)
    pltpu.async_copy(table_hbm.at[mb[0].at[pl.ds(0, C)]], rows[0], semg[0])
    wait_meta(1)
    pltpu.async_copy(table_hbm.at[mb[1].at[pl.ds(0, C)]], rows[1], semg[1])

    def oct_(q, carry):
        for b8 in range(8):
            ci = 8 * q + b8
            b4 = b8 % 4
            rv = rows[b4]
            pltpu.make_async_copy(table_hbm.at[mb[b8].at[pl.ds(0, C)]], rv,
                                  semg[b4]).wait()

            def grp(g, c2):
                wv = plsc.bitcast(mb[b8][pl.ds(64 + g * L, L)], F32)
                dvx[b8][pl.ds(g * L, L)] = mb[b8][pl.ds(128 + g * L, L)]
                for j in range(L):
                    wj = _splat(wv, j)
                    row = g * L + j
                    for k in range(H // L):
                        s = pl.ds(k * L, L)
                        rv[row, s] = rv[row, s] * wj
                return c2

            lax.fori_loop(0, C // L, grp, 0)
            pltpu.async_copy(rv, acc_sh.at[dvx[b8]], sems[b4], add=True)

            @pl.when(ci + 3 < NCH)
            def _():
                issue_meta(ci + 3, (b8 + 3) % 8)

            @pl.when(ci + 2 < NCH)
            def _():
                wait_meta((b8 + 2) % 8)

                @pl.when(ci >= 2)
                def _():
                    pltpu.make_async_copy(rows[(b4 + 2) % 4],
                                          acc_sh.at[dvx[(b8 + 2) % 8]],
                                          sems[(b4 + 2) % 4]).wait()

                pltpu.async_copy(
                    table_hbm.at[mb[(b8 + 2) % 8].at[pl.ds(0, C)]],
                    rows[(b4 + 2) % 4], semg[(b4 + 2) % 4])
        return carry

    lax.fori_loop(0, NCH // 8, oct_, 0)
    for b in range(4):
        pltpu.make_async_copy(rows[b], acc_sh.at[dvx[4 + b]],
                              sems[b]).wait()
    plsc.subcore_barrier()
    pltpu.sync_copy(acc_sh.at[pl.ds(sid * NPR, NPR)],
                    out_hbm.at[pl.ds(cid * NP + sid * NPR, NPR)])


def _make_edge(NP, H, EP):
    EPW = EP // NW
    C = 64
    NCH = EPW // C
    assert EPW % C == 0 and NP % (NS * C) == 0 and NCH % 8 == 0
    mesh = plsc.VectorSubcoreMesh(core_axis_name="c", subcore_axis_name="s",
                                  num_cores=NC, num_subcores=NS)
    return pl.kernel(
        functools.partial(_edge_body, NP, H, EP, EPW, C),
        out_type=jax.ShapeDtypeStruct((NC * NP, H), F32),
        mesh=mesh,
        compiler_params=pltpu.CompilerParams(needs_layout_passes=False),
        scratch_types=(
            [pltpu.VMEM((3 * C,), I32) for _ in range(8)]
            + [pltpu.VMEM((C,), I32) for _ in range(8)]
            + [pltpu.VMEM((C, H), F32) for _ in range(4)]
            + [pltpu.VMEM_SHARED((NP, H), F32)]
            + [pltpu.SemaphoreType.DMA for _ in range(16)]
        ),
    )


def _tgather_body(H, NTP, TPW, CT, tidx_hbm, self_hbm, p0_hbm, p1_hbm,
                  out_hbm, idx_v, a_v, b_v, c_v, sem):
    cid = lax.axis_index("c")
    sid = lax.axis_index("s")
    wid = cid * NS + sid
    tbase = wid * TPW

    def chunk(ci, carry):
        base = tbase + ci * CT
        pltpu.sync_copy(tidx_hbm.at[pl.ds(base, CT)], idx_v)
        ca = pltpu.async_copy(self_hbm.at[idx_v], a_v, sem)
        cb = pltpu.async_copy(p0_hbm.at[idx_v], b_v, sem)
        cc = pltpu.async_copy(p1_hbm.at[idx_v], c_v, sem)
        ca.wait()
        cb.wait()
        cc.wait()

        def row(j, c2):
            for k in range(H // L):
                s = pl.ds(k * L, L)
                a_v[j, s] = jnp.maximum(a_v[j, s] + b_v[j, s] + c_v[j, s],
                                        jnp.zeros((L,), F32))
            return c2

        lax.fori_loop(0, CT, row, 0)
        pltpu.sync_copy(a_v, out_hbm.at[pl.ds(base, CT)])
        return carry

    lax.fori_loop(0, TPW // CT, chunk, 0)


def _make_tgather(N, H, NTP):
    TPW = NTP // NW
    CT = 80
    assert TPW % CT == 0
    mesh = plsc.VectorSubcoreMesh(core_axis_name="c", subcore_axis_name="s",
                                  num_cores=NC, num_subcores=NS)
    return pl.kernel(
        functools.partial(_tgather_body, H, NTP, TPW, CT),
        out_type=jax.ShapeDtypeStruct((NTP, H), F32),
        mesh=mesh,
        scratch_types=[
            pltpu.VMEM((CT,), I32),
            pltpu.VMEM((CT, H), F32),
            pltpu.VMEM((CT, H), F32),
            pltpu.VMEM((CT, H), F32),
            pltpu.SemaphoreType.DMA,
        ],
    )


def _expand_front_input(NTYPES, refs):
    (nt_ref, x_ref, wxT_ref, wtT_ref, temb_ref, bin_ref) = refs
    tb = jnp.dot(temb_ref[...], wtT_ref[...], preferred_element_type=F32)
    nt = nt_ref[...]
    tsel = jnp.zeros((nt.shape[0], tb.shape[1]), F32)
    for t in range(NTYPES):
        tsel = tsel + jnp.where(nt == t, 1.0, 0.0) * tb[t][None, :]
    x = x_ref[...]
    return jnp.maximum(
        jnp.dot(x, wxT_ref[...], preferred_element_type=F32)
        + tsel + bin_ref[...], 0.0)


def _expand_body_common(R, rel_ref, h, selfW_ref, selfb_ref, W2_ref,
                        table_ref, selfp_ref):
    table_ref[...] = jnp.dot(h, W2_ref[...], preferred_element_type=F32)
    Wc = jnp.zeros(selfW_ref.shape[1:], F32)
    bc = jnp.zeros((1, selfb_ref.shape[1]), F32)
    for r in range(R):
        act = rel_ref[r + 1] > rel_ref[r]
        Wc = Wc + jnp.where(act, selfW_ref[r], 0.0)
        bc = bc + jnp.where(act, selfb_ref[r][None, :], 0.0)
    selfp_ref[...] = lax.dot_general(
        h, Wc, (((1,), (1,)), ((), ())), preferred_element_type=F32) + bc


def _tca_body(R, NTYPES, rel_ref, nt_ref, x_ref, wxT_ref, wtT_ref, temb_ref,
              bin_ref, selfW_ref, selfb_ref, W2_ref, table_ref, selfp_ref):
    h = _expand_front_input(
        NTYPES, (nt_ref, x_ref, wxT_ref, wtT_ref, temb_ref, bin_ref))
    _expand_body_common(R, rel_ref, h, selfW_ref, selfb_ref, W2_ref,
                        table_ref, selfp_ref)


def _tcb_body(R, rel_ref, sp_ref, p0_ref, p1_ref, selfW_ref, selfb_ref,
              W2_ref, table_ref, selfp_ref):
    h = jnp.maximum(sp_ref[...] + p0_ref[...] + p1_ref[...], 0.0)
    _expand_body_common(R, rel_ref, h, selfW_ref, selfb_ref, W2_ref,
                        table_ref, selfp_ref)


def _tcd_body(ht_ref, w1_ref, b1_ref, w2_ref, b2_ref, out_ref):
    m = jnp.maximum(
        jnp.dot(ht_ref[...], w1_ref[...], preferred_element_type=F32)
        + b1_ref[...], 0.0)
    out_ref[...] = jnp.dot(m, w2_ref[...], preferred_element_type=F32) \
        + b2_ref[...]


def kernel(x, edge_index, edge_weight, rel_ptr, node_type, target_index,
           type_emb, lin_in_W, lin_in_b, l1_self_W, l1_self_b, l1_nei_W,
           l2_self_W, l2_self_b, l2_nei_W, mlp1_W, mlp1_b, mlp2_W, mlp2_b):
    N, D = x.shape
    E = edge_index.shape[1]
    R, H, _ = l1_self_W.shape
    NTYPES, TE = type_emb.shape
    OUT = mlp2_W.shape[0]
    NT = target_index.shape[0]

    EP = ((E + NW * 512 - 1) // (NW * 512)) * (NW * 512)
    NTP = ((NT + NW * 80 - 1) // (NW * 80)) * (NW * 80)

    src = jnp.concatenate([edge_index[0], jnp.zeros((EP - E,), I32)])
    dstp = jnp.concatenate([edge_index[1], jnp.zeros((EP - E,), I32)])
    ewp = jnp.concatenate([edge_weight, jnp.zeros((EP - E,), F32)])
    relpad = jnp.concatenate([rel_ptr, jnp.zeros((16 - (R + 1),), I32)])
    tidxp = jnp.concatenate([target_index, jnp.zeros((NTP - NT,), I32)])
    nt2 = node_type.reshape(N, 1)
    wxT = lin_in_W[:, :D].T
    wtT = lin_in_W[:, D:].T
    bin2 = lin_in_b.reshape(1, H)
    W2_1 = l1_nei_W.transpose(2, 0, 1).reshape(H, R * H)
    W2_2 = l2_nei_W.transpose(2, 0, 1).reshape(H, R * H)
    w1T = mlp1_W.T
    b1r = mlp1_b.reshape(1, H)
    w2T = mlp2_W.T
    b2r = mlp2_b.reshape(1, OUT)

    meta = _make_prep(R, EP)(src, dstp, ewp, relpad)

    BN = 1000
    grid = (N // BN,)
    tca = pl.pallas_call(
        functools.partial(_tca_body, R, NTYPES),
        grid=grid,
        in_specs=[
            pl.BlockSpec(memory_space=pltpu.SMEM),
            pl.BlockSpec((BN, 1), lambda i: (i, 0)),
            pl.BlockSpec((BN, D), lambda i: (i, 0)),
            pl.BlockSpec((D, H), lambda i: (0, 0)),
            pl.BlockSpec((TE, H), lambda i: (0, 0)),
            pl.BlockSpec((NTYPES, TE), lambda i: (0, 0)),
            pl.BlockSpec((1, H), lambda i: (0, 0)),
            pl.BlockSpec((R, H, H), lambda i: (0, 0, 0)),
            pl.BlockSpec((R, H), lambda i: (0, 0)),
            pl.BlockSpec((H, R * H), lambda i: (0, 0)),
        ],
        out_specs=[
            pl.BlockSpec((BN, R * H), lambda i: (i, 0)),
            pl.BlockSpec((BN, H), lambda i: (i, 0)),
        ],
        out_shape=[
            jax.ShapeDtypeStruct((N, R * H), F32),
            jax.ShapeDtypeStruct((N, H), F32),
        ],
    )
    table1, self1 = tca(rel_ptr, nt2, x, wxT, wtT, type_emb, bin2,
                        l1_self_W, l1_self_b, W2_1)

    NPAD = ((N + NS * 128 - 1) // (NS * 128)) * (NS * 128)
    edge_k = _make_edge(NPAD, H, EP)
    parts1 = edge_k(table1.reshape(N * R, H), meta)
    p1a, p1b = parts1[:N], parts1[NPAD:NPAD + N]

    tcb = pl.pallas_call(
        functools.partial(_tcb_body, R),
        grid=grid,
        in_specs=[
            pl.BlockSpec(memory_space=pltpu.SMEM),
            pl.BlockSpec((BN, H), lambda i: (i, 0)),
            pl.BlockSpec((BN, H), lambda i: (i, 0)),
            pl.BlockSpec((BN, H), lambda i: (i, 0)),
            pl.BlockSpec((R, H, H), lambda i: (0, 0, 0)),
            pl.BlockSpec((R, H), lambda i: (0, 0)),
            pl.BlockSpec((H, R * H), lambda i: (0, 0)),
        ],
        out_specs=[
            pl.BlockSpec((BN, R * H), lambda i: (i, 0)),
            pl.BlockSpec((BN, H), lambda i: (i, 0)),
        ],
        out_shape=[
            jax.ShapeDtypeStruct((N, R * H), F32),
            jax.ShapeDtypeStruct((N, H), F32),
        ],
    )
    table2, self2 = tcb(rel_ptr, self1, p1a, p1b,
                        l2_self_W, l2_self_b, W2_2)

    parts2 = edge_k(table2.reshape(N * R, H), meta)
    p2a, p2b = parts2[:N], parts2[NPAD:NPAD + N]

    htp = _make_tgather(N, H, NTP)(tidxp, self2, p2a, p2b)
    ht = htp[:NT]

    BM = 200
    tcd = pl.pallas_call(
        _tcd_body,
        grid=(NT // BM,),
        in_specs=[
            pl.BlockSpec((BM, H), lambda i: (i, 0)),
            pl.BlockSpec((H, H), lambda i: (0, 0)),
            pl.BlockSpec((1, H), lambda i: (0, 0)),
            pl.BlockSpec((H, OUT), lambda i: (0, 0)),
            pl.BlockSpec((1, OUT), lambda i: (0, 0)),
        ],
        out_specs=pl.BlockSpec((BM, OUT), lambda i: (i, 0)),
        out_shape=jax.ShapeDtypeStruct((NT, OUT), F32),
    )
    return tcd(ht, w1T, b1r, w2T, b2r)
